# Initial kernel scaffold; baseline (speedup 1.0000x reference)
import jax
import jax.numpy as jnp
from jax.experimental import pallas as pl


def kernel(x, Wr, W1, b1, W2, b2):
    raise NotImplementedError("write your pallas kernel here")


# speedup vs baseline: 1.9650x; 1.9650x over previous
import functools
import jax
import jax.numpy as jnp
from jax.experimental import pallas as pl
from jax.experimental.pallas import tpu as pltpu

S, D, H, E, TOPK = 2048, 768, 768, 8, 2
TT = 512


def _top2_gates(logits):
    U = (jax.lax.broadcasted_iota(jnp.int32, (E, E), 0)
         <= jax.lax.broadcasted_iota(jnp.int32, (E, E), 1)).astype(jnp.float32)
    m1 = jnp.max(logits, axis=-1, keepdims=True)
    is1 = (logits == m1).astype(jnp.float32)
    first1 = is1 * (jnp.dot(is1, U, preferred_element_type=jnp.float32) == 1.0)
    masked = jnp.where(first1 > 0, -jnp.inf, logits)
    m2 = jnp.max(masked, axis=-1, keepdims=True)
    is2 = (masked == m2).astype(jnp.float32)
    first2 = is2 * (jnp.dot(is2, U, preferred_element_type=jnp.float32) == 1.0)
    g1 = jax.nn.sigmoid(m1 - m2)
    g2 = 1.0 - g1
    return g1, g2, first1, first2


def _gelu_exact(x):
    return 0.5 * x * (1.0 + jax.lax.erf(x * jnp.float32(0.7071067811865476)))


def _dense_body(x_ref, wr_ref, w1_ref, b1_ref, w2_ref, b2_ref, o_ref):
    e = pl.program_id(1)
    xb = x_ref[...]
    logits = jnp.dot(xb, wr_ref[...], preferred_element_type=jnp.float32)
    g1, g2, first1, first2 = _top2_gates(logits)
    gates = g1 * first1 + g2 * first2
    oe = (jax.lax.broadcasted_iota(jnp.int32, (E, 1), 0) == e).astype(jnp.float32)
    w_col = jnp.dot(gates, oe, preferred_element_type=jnp.float32)

    h = jnp.dot(xb, w1_ref[0], preferred_element_type=jnp.float32) + b1_ref[0]
    h = _gelu_exact(h)
    o = jnp.dot(h, w2_ref[0], preferred_element_type=jnp.float32) + b2_ref[0]
    o = o * w_col

    @pl.when(e == 0)
    def _():
        o_ref[...] = o

    @pl.when(e > 0)
    def _():
        o_ref[...] = o_ref[...] + o


@jax.jit
def _moe_fused(x2, Wr, W1, b1, W2, b2):
    return pl.pallas_call(
        _dense_body,
        grid=(S // TT, E),
        in_specs=[
            pl.BlockSpec((TT, D), lambda t, e: (t, 0)),
            pl.BlockSpec((D, E), lambda t, e: (0, 0)),
            pl.BlockSpec((1, D, H), lambda t, e: (e, 0, 0)),
            pl.BlockSpec((1, 1, H), lambda t, e: (e, 0, 0)),
            pl.BlockSpec((1, H, D), lambda t, e: (e, 0, 0)),
            pl.BlockSpec((1, 1, D), lambda t, e: (e, 0, 0)),
        ],
        out_specs=pl.BlockSpec((TT, D), lambda t, e: (t, 0)),
        out_shape=jax.ShapeDtypeStruct((S, D), jnp.float32),
        compiler_params=pltpu.CompilerParams(
            dimension_semantics=("arbitrary", "arbitrary"),
        ),
    )(x2, Wr, W1, b1.reshape(E, 1, H), W2, b2.reshape(E, 1, D))


def kernel(x, Wr, W1, b1, W2, b2):
    orig_shape = x.shape
    x2 = x.reshape(-1, x.shape[-1])
    out = _moe_fused(x2, Wr, W1, b1, W2, b2)
    return out.reshape(orig_shape)
